# Initial kernel scaffold; baseline (speedup 1.0000x reference)
import jax
import jax.numpy as jnp
from jax.experimental import pallas as pl


def kernel(output, target, months):
    raise NotImplementedError("write your pallas kernel here")


# speedup vs baseline: 126.9046x; 126.9046x over previous
import jax
import jax.numpy as jnp
from jax.experimental import pallas as pl
from jax.experimental.pallas import tpu as pltpu

_N = 1048576
_M = 12
_RAIN = 0.1
_LANES = 128
_ROWS = _N // _LANES
_G = 8
_R = _ROWS // _G


def _body(xo_ref, xt_ref, mo_ref, out_ref, acc_ref):
    step = pl.program_id(0)

    @pl.when(step == 0)
    def _():
        acc_ref[...] = jnp.zeros_like(acc_ref)

    xo = xo_ref[...]
    xt = xt_ref[...]
    mo = mo_ref[...]
    ro = jnp.where(xo < _RAIN, 0.0, 1.0).astype(jnp.float32)
    rt = jnp.where(xt < _RAIN, 0.0, 1.0).astype(jnp.float32)
    wo = xo * ro
    wt = xt * rt

    def gsum(a):
        return a.reshape(_R // 8, 8, _LANES).sum(axis=0)

    for m in range(_M):
        cm = mo == m
        zf = jnp.float32(0.0)
        sel_o = jnp.where(cm, wo, zf)
        sel_t = jnp.where(cm, wt, zf)
        sel_ro = jnp.where(cm, ro, zf)
        sel_rt = jnp.where(cm, rt, zf)
        cf = jnp.where(cm, 1.0, 0.0).astype(jnp.float32)
        b = m * 7
        acc_ref[b + 0] += gsum(sel_o)
        acc_ref[b + 1] += gsum(sel_o * sel_o)
        acc_ref[b + 2] += gsum(sel_ro)
        acc_ref[b + 3] += gsum(sel_t)
        acc_ref[b + 4] += gsum(sel_t * sel_t)
        acc_ref[b + 5] += gsum(sel_rt)
        acc_ref[b + 6] += gsum(cf)

    @pl.when(step == _G - 1)
    def _():
        total = jnp.float32(0.0)
        for m in range(_M):
            b = m * 7
            s1o = jnp.sum(acc_ref[b + 0])
            s2o = jnp.sum(acc_ref[b + 1])
            cro = jnp.sum(acc_ref[b + 2])
            s1t = jnp.sum(acc_ref[b + 3])
            s2t = jnp.sum(acc_ref[b + 4])
            crt = jnp.sum(acc_ref[b + 5])
            cnt = jnp.sum(acc_ref[b + 6])
            pos = cnt > 0
            mu_o = jnp.where(pos, s1o / cnt, 0.0)
            mu_t = jnp.where(pos, s1t / cnt, 0.0)
            vo = s2o - 2.0 * mu_o * s1o + mu_o * mu_o * cro
            vt = s2t - 2.0 * mu_t * s1t + mu_t * mu_t * crt
            vo = jnp.where(pos, vo / cnt, 0.0)
            vt = jnp.where(pos, vt / cnt, 0.0)
            so = jnp.sqrt(jnp.maximum(vo, 0.0))
            st = jnp.sqrt(jnp.maximum(vt, 0.0))
            d = so - st
            total = total + d * d
        out_ref[...] = (total / _M).reshape(1, 1)


@jax.jit
def kernel(output, target, months):
    xo = output.reshape(_ROWS, _LANES)
    xt = target.reshape(_ROWS, _LANES)
    mo = months.reshape(_ROWS, _LANES)
    out = pl.pallas_call(
        _body,
        grid=(_G,),
        in_specs=[
            pl.BlockSpec((_R, _LANES), lambda i: (i, 0)),
            pl.BlockSpec((_R, _LANES), lambda i: (i, 0)),
            pl.BlockSpec((_R, _LANES), lambda i: (i, 0)),
        ],
        out_specs=pl.BlockSpec((1, 1), lambda i: (0, 0)),
        out_shape=jax.ShapeDtypeStruct((1, 1), jnp.float32),
        scratch_shapes=[pltpu.VMEM((7 * _M, 8, _LANES), jnp.float32)],
    )(xo, xt, mo)
    return out[0, 0]


# speedup vs baseline: 129.0312x; 1.0168x over previous
import jax
import jax.numpy as jnp
from jax.experimental import pallas as pl
from jax.experimental.pallas import tpu as pltpu

_N = 1048576
_M = 12
_NQ = 7 * _M
_RAIN = 0.1
_LANES = 128
_ROWS = _N // _LANES
_CORES = 2
_G = 8
_R = _ROWS // (_CORES * _G)


def _body(xo_ref, xt_ref, mo_ref, out_ref, acc_ref):
    i = pl.program_id(1)

    @pl.when(i == 0)
    def _():
        acc_ref[...] = jnp.zeros_like(acc_ref)

    xo = xo_ref[...]
    xt = xt_ref[...]
    mo = mo_ref[...]
    ro = (xo >= _RAIN).astype(jnp.float32)
    rt = (xt >= _RAIN).astype(jnp.float32)
    wo = xo * ro
    wt = xt * rt

    def gsum(a):
        return a.reshape(_R // 8, 8, _LANES).sum(axis=0)

    for m in range(_M):
        cf = (mo == m).astype(jnp.float32)
        p_o = cf * wo
        p_t = cf * wt
        b = m * 7
        acc_ref[b + 0] += gsum(p_o)
        acc_ref[b + 1] += gsum(p_o * wo)
        acc_ref[b + 2] += gsum(cf * ro)
        acc_ref[b + 3] += gsum(p_t)
        acc_ref[b + 4] += gsum(p_t * wt)
        acc_ref[b + 5] += gsum(cf * rt)
        acc_ref[b + 6] += gsum(cf)

    @pl.when(i == _G - 1)
    def _():
        lane = jax.lax.broadcasted_iota(jnp.int32, (1, _LANES), 1)
        row = jnp.zeros((1, _LANES), jnp.float32)
        for p in range(_NQ):
            s = jnp.sum(acc_ref[p])
            row = row + jnp.where(lane == p, s, 0.0)
        out_ref[...] = row.reshape(1, 1, _LANES)


@jax.jit
def kernel(output, target, months):
    xo = output.reshape(_ROWS, _LANES)
    xt = target.reshape(_ROWS, _LANES)
    mo = months.reshape(_ROWS, _LANES)
    partials = pl.pallas_call(
        _body,
        grid=(_CORES, _G),
        in_specs=[
            pl.BlockSpec((_R, _LANES), lambda c, i: (c * _G + i, 0)),
            pl.BlockSpec((_R, _LANES), lambda c, i: (c * _G + i, 0)),
            pl.BlockSpec((_R, _LANES), lambda c, i: (c * _G + i, 0)),
        ],
        out_specs=pl.BlockSpec((1, 1, _LANES), lambda c, i: (c, 0, 0)),
        out_shape=jax.ShapeDtypeStruct((_CORES, 1, _LANES), jnp.float32),
        scratch_shapes=[pltpu.VMEM((_NQ, 8, _LANES), jnp.float32)],
        compiler_params=pltpu.CompilerParams(
            dimension_semantics=("parallel", "arbitrary"),
        ),
    )(xo, xt, mo)

    t = partials[:, 0, :_NQ].sum(axis=0).reshape(_M, 7)
    s1o, s2o, cro = t[:, 0], t[:, 1], t[:, 2]
    s1t, s2t, crt = t[:, 3], t[:, 4], t[:, 5]
    cnt = t[:, 6]
    pos = cnt > 0
    mu_o = jnp.where(pos, s1o / cnt, 0.0)
    mu_t = jnp.where(pos, s1t / cnt, 0.0)
    vo = s2o - 2.0 * mu_o * s1o + mu_o * mu_o * cro
    vt = s2t - 2.0 * mu_t * s1t + mu_t * mu_t * crt
    vo = jnp.where(pos, vo / cnt, 0.0)
    vt = jnp.where(pos, vt / cnt, 0.0)
    so = jnp.sqrt(jnp.maximum(vo, 0.0))
    st = jnp.sqrt(jnp.maximum(vt, 0.0))
    return jnp.mean((so - st) ** 2)
